# Initial kernel scaffold; baseline (speedup 1.0000x reference)
import jax
import jax.numpy as jnp
from jax.experimental import pallas as pl


def kernel(x, table):
    raise NotImplementedError("write your pallas kernel here")


# speedup vs baseline: 1.8447x; 1.8447x over previous
import functools

import jax
import jax.numpy as jnp
from jax import lax
from jax.experimental import pallas as pl
from jax.experimental.pallas import tpu as pltpu
from jax.experimental.pallas import tpu_sc as plsc

_D = 64
_NC = 2
_NS = 16
_NW = _NC * _NS
_IW = 128
_K = 8
_G = _K * _IW


def _make_gather(tot: int):
    per_w = tot // _NW
    groups = per_w // _G
    rows_per_w = per_w // _IW

    mesh = plsc.VectorSubcoreMesh(core_axis_name="c", subcore_axis_name="s")

    @functools.partial(
        pl.kernel,
        out_type=jax.ShapeDtypeStruct((tot, _D), jnp.float32),
        mesh=mesh,
        scratch_types=[
            pltpu.VMEM((_K, _IW), jnp.int32),
            pltpu.VMEM((_G, _D), jnp.float32),
            pltpu.SemaphoreType.DMA,
        ],
        compiler_params=pltpu.CompilerParams(use_tc_tiling_on_sc=False),
    )
    def gather(x_hbm, tab_hbm, out_hbm, idx_v, rows_v, sem):
        wid = lax.axis_index("s") * _NC + lax.axis_index("c")
        idx_row0 = wid * rows_per_w
        out0 = wid * per_w

        def group_fn(g, carry):
            pltpu.sync_copy(x_hbm.at[pl.ds(idx_row0 + g * _K, _K)], idx_v)
            copies = []
            for j in range(_K):
                copies.append(
                    pltpu.async_copy(
                        tab_hbm.at[idx_v.at[j]],
                        rows_v.at[pl.ds(j * _IW, _IW)],
                        sem,
                    )
                )
            for cp in copies:
                cp.wait()
            pltpu.sync_copy(rows_v, out_hbm.at[pl.ds(out0 + g * _G, _G)])
            return carry

        lax.fori_loop(0, groups, group_fn, 0)

    return gather


def kernel(x, table):
    tot = x.size
    xf = x.reshape(tot // _IW, _IW).astype(jnp.int32)
    out = _make_gather(tot)(xf, table)
    return out.reshape(x.shape + (table.shape[1],))


# speedup vs baseline: 1.8551x; 1.0056x over previous
import functools

import jax
import jax.numpy as jnp
from jax import lax
from jax.experimental import pallas as pl
from jax.experimental.pallas import tpu as pltpu
from jax.experimental.pallas import tpu_sc as plsc

_D = 64
_NC = 2
_NS = 16
_NW = _NC * _NS
_IW = 128
_K = 4
_G = _K * _IW
_NBUF = 2


def _make_gather(tot: int):
    per_w = tot // _NW
    groups = per_w // _G
    rows_per_w = per_w // _IW
    assert groups % _NBUF == 0 and groups >= 2 * _NBUF

    mesh = plsc.VectorSubcoreMesh(core_axis_name="c", subcore_axis_name="s")

    @functools.partial(
        pl.kernel,
        out_type=jax.ShapeDtypeStruct((tot, _D), jnp.float32),
        mesh=mesh,
        scratch_types=[
            pltpu.VMEM((_NBUF, _K, _IW), jnp.int32),
            pltpu.VMEM((_NBUF, _G, _D), jnp.float32),
            [pltpu.SemaphoreType.DMA] * _NBUF,
            [pltpu.SemaphoreType.DMA] * _NBUF,
        ],
        compiler_params=pltpu.CompilerParams(use_tc_tiling_on_sc=False),
    )
    def gather(x_hbm, tab_hbm, out_hbm, idx_v, rows_v, sg, sw):
        wid = lax.axis_index("s") * _NC + lax.axis_index("c")
        irow0 = wid * rows_per_w
        out0 = wid * per_w

        def issue(g, b):
            pltpu.sync_copy(x_hbm.at[pl.ds(irow0 + g * _K, _K)], idx_v.at[b])
            for j in range(_K):
                pltpu.async_copy(
                    tab_hbm.at[idx_v.at[b, j]],
                    rows_v.at[b, pl.ds(j * _IW, _IW)],
                    sg[b],
                )

        def wait_gathers(b):
            for j in range(_K):
                pltpu.make_async_copy(
                    tab_hbm.at[idx_v.at[b, j]],
                    rows_v.at[b, pl.ds(j * _IW, _IW)],
                    sg[b],
                ).wait()

        def start_wb(g, b):
            pltpu.async_copy(rows_v.at[b], out_hbm.at[pl.ds(out0 + g * _G, _G)], sw[b])

        def wait_wb(g, b):
            pltpu.make_async_copy(
                rows_v.at[b], out_hbm.at[pl.ds(out0 + g * _G, _G)], sw[b]
            ).wait()

        for b in range(_NBUF):
            issue(b, b)

        @pl.loop(0, groups - _NBUF, step=_NBUF)
        def _(g):
            for b in range(_NBUF):
                gg = g + b
                wait_gathers(b)
                start_wb(gg, b)
                wait_wb(gg, b)
                issue(gg + _NBUF, b)

        for b in range(_NBUF):
            gg = groups - _NBUF + b
            wait_gathers(b)
            start_wb(gg, b)
            wait_wb(gg, b)

    return gather


def kernel(x, table):
    tot = x.size
    xf = x.reshape(tot // _IW, _IW).astype(jnp.int32)
    out = _make_gather(tot)(xf, table)
    return out.reshape(x.shape + (table.shape[1],))
